# baseline (speedup 1.0000x reference)
import jax
import jax.numpy as jnp
from jax.experimental import pallas as pl
from jax.experimental.pallas import tpu as pltpu

MAX_LEN = 8192
DIM = 1024
BLOCK_ROWS = 2048


def _body(t_ref, pe_ref, out_ref, masked_buf, sems):
    i = pl.program_id(0)
    t = t_ref[0]
    blk_start = i * BLOCK_ROWS
    dst = out_ref.at[pl.ds(blk_start, BLOCK_ROWS)]

    @pl.when(blk_start + BLOCK_ROWS <= t)
    def _full_copy():
        c = pltpu.make_async_copy(pe_ref, dst, sems.at[0])
        c.start()
        c.wait()

    @pl.when(blk_start + BLOCK_ROWS > t)
    def _masked_copy():
        rows = jax.lax.broadcasted_iota(jnp.int32, (BLOCK_ROWS, 1), 0) + blk_start
        masked_buf[...] = jnp.where(rows < t, pe_ref[...], 0.0)
        c = pltpu.make_async_copy(masked_buf, dst, sems.at[1])
        c.start()
        c.wait()


def kernel(pe, T):
    t_arr = jnp.asarray(T, dtype=jnp.int32).reshape((1,))
    n_blocks = MAX_LEN // BLOCK_ROWS
    out = pl.pallas_call(
        _body,
        grid=(n_blocks,),
        in_specs=[
            pl.BlockSpec(memory_space=pltpu.SMEM),
            pl.BlockSpec((BLOCK_ROWS, DIM), lambda i: (i, 0)),
        ],
        out_specs=pl.BlockSpec(memory_space=pl.ANY),
        out_shape=jax.ShapeDtypeStruct((MAX_LEN, DIM), jnp.float32),
        scratch_shapes=[
            pltpu.VMEM((BLOCK_ROWS, DIM), jnp.float32),
            pltpu.SemaphoreType.DMA((2,)),
        ],
    )(t_arr, pe)
    return out[None, :, :]


# speedup vs baseline: 1.0314x; 1.0314x over previous
import jax
import jax.numpy as jnp
from jax.experimental import pallas as pl
from jax.experimental.pallas import tpu as pltpu

MAX_LEN = 8192
DIM = 1024
BLOCK_ROWS = 2048


def _body(t_ref, pe_ref, out_ref):
    i = pl.program_id(0)
    t = t_ref[0]
    blk_start = i * BLOCK_ROWS

    @pl.when(blk_start + BLOCK_ROWS <= t)
    def _full_copy():
        out_ref[...] = pe_ref[...]

    @pl.when(blk_start + BLOCK_ROWS > t)
    def _masked_copy():
        rows = jax.lax.broadcasted_iota(jnp.int32, (BLOCK_ROWS, 1), 0) + blk_start
        out_ref[...] = jnp.where(rows < t, pe_ref[...], 0.0)


def kernel(pe, T):
    t_arr = jnp.asarray(T, dtype=jnp.int32).reshape((1,))
    n_blocks = MAX_LEN // BLOCK_ROWS
    out = pl.pallas_call(
        _body,
        grid=(n_blocks,),
        in_specs=[
            pl.BlockSpec(memory_space=pltpu.SMEM),
            pl.BlockSpec((BLOCK_ROWS, DIM), lambda i: (i, 0)),
        ],
        out_specs=pl.BlockSpec((BLOCK_ROWS, DIM), lambda i: (i, 0)),
        out_shape=jax.ShapeDtypeStruct((MAX_LEN, DIM), jnp.float32),
    )(t_arr, pe)
    return out[None, :, :]
